# baseline (speedup 1.0000x reference)
import functools

import jax
import jax.numpy as jnp
from jax.experimental import pallas as pl
from jax.experimental.pallas import tpu as pltpu

_VMEM_LIMIT = 48 * 1024 * 1024
_N_TILE = 8192


def _pool_body(vals_ref, mask_ref, out_ref, acc_ref, cnt_ref):
    k = pl.program_id(1)
    v = vals_ref[0]
    m = mask_ref[0].astype(jnp.float32)
    s = jnp.dot(m, v, preferred_element_type=jnp.float32)
    cnt = jnp.sum(m)

    @pl.when(k == 0)
    def _init():
        acc_ref[...] = s
        cnt_ref[0] = cnt

    @pl.when(k != 0)
    def _accum():
        acc_ref[...] += s
        cnt_ref[0] += cnt

    @pl.when(k == pl.num_programs(1) - 1)
    def _finalize():
        inv = 1.0 / jnp.maximum(cnt_ref[0], 1.0)
        out_ref[0] = acc_ref[...] * inv


def kernel(coords, vals, mask):
    del coords
    bs, n, c = vals.shape
    mask3 = mask.reshape(bs, 1, n)
    n_tile = _N_TILE if n % _N_TILE == 0 else n
    n_tiles = n // n_tile

    cost = pl.CostEstimate(
        flops=2 * bs * n * c, transcendentals=0,
        bytes_accessed=bs * n * c * 4 + bs * n + bs * c * 4)

    out = pl.pallas_call(
        _pool_body,
        out_shape=jax.ShapeDtypeStruct((bs, 1, c), jnp.float32),
        grid=(bs, n_tiles),
        in_specs=[
            pl.BlockSpec((1, n_tile, c), lambda b, k: (b, k, 0)),
            pl.BlockSpec((1, 1, n_tile), lambda b, k: (b, 0, k)),
        ],
        out_specs=pl.BlockSpec((1, 1, c), lambda b, k: (b, 0, 0)),
        scratch_shapes=[pltpu.VMEM((1, c), jnp.float32),
                        pltpu.SMEM((1,), jnp.float32)],
        compiler_params=pltpu.CompilerParams(
            dimension_semantics=("parallel", "arbitrary"),
            vmem_limit_bytes=_VMEM_LIMIT),
        cost_estimate=cost,
    )(vals, mask3)
    return out.reshape(bs, c)


# speedup vs baseline: 1.1106x; 1.1106x over previous
import jax
import jax.numpy as jnp
from jax.experimental import pallas as pl
from jax.experimental.pallas import tpu as pltpu

_VMEM_LIMIT = 48 * 1024 * 1024


def _pool_body(vals_ref, mask_ref, out_ref):
    v = vals_ref[0]
    m = mask_ref[0].astype(jnp.float32)
    s = jnp.dot(m, v, preferred_element_type=jnp.float32)
    cnt = jnp.sum(m)
    inv = 1.0 / jnp.maximum(cnt, 1.0)
    out_ref[0] = s * inv


def kernel(coords, vals, mask):
    del coords
    bs, n, c = vals.shape
    mask3 = mask.reshape(bs, 1, n)

    cost = pl.CostEstimate(
        flops=2 * bs * n * c, transcendentals=0,
        bytes_accessed=bs * n * c * 4 + bs * n + bs * c * 4)

    out = pl.pallas_call(
        _pool_body,
        out_shape=jax.ShapeDtypeStruct((bs, 1, c), jnp.float32),
        grid=(bs,),
        in_specs=[
            pl.BlockSpec((1, n, c), lambda b: (b, 0, 0)),
            pl.BlockSpec((1, 1, n), lambda b: (b, 0, 0)),
        ],
        out_specs=pl.BlockSpec((1, 1, c), lambda b: (b, 0, 0)),
        compiler_params=pltpu.CompilerParams(
            dimension_semantics=("parallel",),
            vmem_limit_bytes=_VMEM_LIMIT),
        cost_estimate=cost,
    )(vals, mask3)
    return out.reshape(bs, c)
